# baseline (speedup 1.0000x reference)
import jax
import jax.numpy as jnp
import numpy as np
from jax import lax
from jax._src.random.threefry2x32 import threefry2x32_p
from jax.experimental import pallas as pl
from jax.experimental.pallas import tpu as pltpu
from jax.experimental.pallas import tpu_sc as plsc

C = 256
T = 500
NT = T + 2
EPS = 1e-6
B, L = 4096, 50
N = B * L
NC, NS = 2, 16
NW = NC * NS
STAGES = 16
NTOK = N // STAGES
NPW = NTOK // NW
CH = 64
NCHUNK = NPW // CH
LANES = 16
JC = C // LANES


def _sc_body(tbl, xs, xe, tt, w, out,
             xs_v, xe_v, tt_v, idx1_v, idx2_v,
             rows_a, rows_b, w_v, out_v,
             sem_a, sem_b, sem_w):
    wid = lax.axis_index("s") * NC + lax.axis_index("c")
    base = wid * NPW

    pltpu.sync_copy(xs.at[pl.ds(base, NPW)], xs_v)
    pltpu.sync_copy(xe.at[pl.ds(base, NPW)], xe_v)
    pltpu.sync_copy(tt.at[pl.ds(base, NPW)], tt_v)

    def idx_body(k, carry):
        sl = pl.ds(k * LANES, LANES)
        tv = tt_v[sl]
        idx1_v[sl] = tv * C + xs_v[sl]
        idx2_v[sl] = (T + 1 - tv) * C + xe_v[sl]
        return carry

    lax.fori_loop(0, NPW // LANES, idx_body, 0)

    lane = lax.iota(jnp.int32, LANES)

    def chunk_body(cidx, carry):
        off = cidx * CH
        ca = pltpu.async_copy(tbl.at[idx1_v.at[pl.ds(off, CH)]], rows_a, sem_a)
        cb = pltpu.async_copy(tbl.at[idx2_v.at[pl.ds(off, CH)]], rows_b, sem_b)
        cw = pltpu.async_copy(w.at[pl.ds((base + off) * C, CH * C)], w_v, sem_w)
        ca.wait()
        cb.wait()
        cw.wait()

        skew = lane * 17

        def group_body(g, gcarry):
            tok = lane + g * LANES

            def j_body(s, jc):
                mx, mi = jc
                jv = (skew + s) & (C - 1)
                a = plsc.load_gather(rows_a, [tok, jv])
                b = plsc.load_gather(rows_b, [tok, jv])
                wv = plsc.load_gather(w_v, [tok * C + jv])
                v = (a + EPS) * (b + EPS) * wv
                upd = v > mx
                mx = jnp.where(upd, v, mx)
                mi = jnp.where(upd, jv, mi)
                return (mx, mi)

            _, mi = lax.fori_loop(
                0, C, j_body,
                (jnp.full((LANES,), -1.0, jnp.float32),
                 jnp.zeros((LANES,), jnp.int32)),
                unroll=8)
            sl = pl.ds(off + g * LANES, LANES)
            out_v[sl] = jnp.where(tt_v[sl] == T + 1, xe_v[sl], mi)
            return gcarry

        lax.fori_loop(0, CH // LANES, group_body, 0)
        return carry

    lax.fori_loop(0, NCHUNK, chunk_body, 0)
    pltpu.sync_copy(out_v, out.at[pl.ds(base, NPW)])


_sc_call = pl.kernel(
    _sc_body,
    out_type=jax.ShapeDtypeStruct((NTOK,), jnp.int32),
    mesh=plsc.VectorSubcoreMesh(core_axis_name="c", subcore_axis_name="s"),
    compiler_params=pltpu.CompilerParams(use_tc_tiling_on_sc=False,
                                         needs_layout_passes=False),
    scratch_types=[
        pltpu.VMEM((NPW,), jnp.int32),
        pltpu.VMEM((NPW,), jnp.int32),
        pltpu.VMEM((NPW,), jnp.int32),
        pltpu.VMEM((NPW,), jnp.int32),
        pltpu.VMEM((NPW,), jnp.int32),
        pltpu.VMEM((CH, C), jnp.float32),
        pltpu.VMEM((CH, C), jnp.float32),
        pltpu.VMEM((CH * C,), jnp.float32),
        pltpu.VMEM((NPW,), jnp.int32),
        pltpu.SemaphoreType.DMA,
        pltpu.SemaphoreType.DMA,
        pltpu.SemaphoreType.DMA,
    ],
)


_K1, _K2 = (int(x) for x in np.asarray(
    jax.random.key_data(jax.random.fold_in(jax.random.key(0), 1))))


def _gumbel_weights(lo, hi):
    c2 = lax.iota(jnp.uint32, hi - lo) + np.uint32(lo)
    b1, b2 = threefry2x32_p.bind(
        jnp.asarray(_K1, jnp.uint32), jnp.asarray(_K2, jnp.uint32),
        jnp.zeros((hi - lo,), jnp.uint32), c2)
    fb = ((b1 ^ b2) >> np.uint32(9)) | np.uint32(0x3F800000)
    u = lax.bitcast_convert_type(fb, jnp.float32) - 1.0
    u = jnp.clip(u, jnp.finfo(jnp.float32).tiny, 1.0)
    return 1.0 / (-jnp.log(u))


def kernel(x_start, x_end, t, p_onestep, p_cum):
    tbl = p_cum.reshape(NT * C, C)
    xs = x_start.reshape(N)
    xe = x_end.reshape(N)
    tt = jnp.repeat(t, L)
    outs = []
    for s in range(STAGES):
        a = s * NTOK
        w_s = _gumbel_weights(a * C, (a + NTOK) * C)
        outs.append(_sc_call(tbl, xs[a:a + NTOK], xe[a:a + NTOK],
                             tt[a:a + NTOK], w_s))
    return jnp.concatenate(outs).reshape(B, L)


# speedup vs baseline: 1.0180x; 1.0180x over previous
import jax
import jax.numpy as jnp
import numpy as np
from jax import lax
from jax._src.random.threefry2x32 import threefry2x32_p
from jax.experimental import pallas as pl
from jax.experimental.pallas import tpu as pltpu
from jax.experimental.pallas import tpu_sc as plsc

C = 256
T = 500
NT = T + 2
EPS = 1e-6
B, L = 4096, 50
N = B * L
NC, NS = 2, 16
NW = NC * NS
STAGES = 8
NTOK = N // STAGES
NPW = NTOK // NW
CH = 80
NCHUNK = NPW // CH
assert NTOK * STAGES == N and NPW * NW == NTOK and NCHUNK * CH == NPW
assert CH % 8 == 0 and CH <= 128
LANES = 16
JC = C // LANES


def _sc_body(tbl, xs, xe, tt, w, out,
             xs_v, xe_v, tt_v, idx1_v, idx2_v,
             rows_a, rows_b, w_v, out_v,
             sem_a, sem_b, sem_w):
    wid = lax.axis_index("s") * NC + lax.axis_index("c")
    base = wid * NPW

    pltpu.sync_copy(xs.at[pl.ds(base, NPW)], xs_v)
    pltpu.sync_copy(xe.at[pl.ds(base, NPW)], xe_v)
    pltpu.sync_copy(tt.at[pl.ds(base, NPW)], tt_v)

    def idx_body(k, carry):
        sl = pl.ds(k * LANES, LANES)
        tv = tt_v[sl]
        idx1_v[sl] = tv * C + xs_v[sl]
        idx2_v[sl] = (T + 1 - tv) * C + xe_v[sl]
        return carry

    lax.fori_loop(0, NPW // LANES, idx_body, 0)

    lane = lax.iota(jnp.int32, LANES)

    def chunk_body(cidx, carry):
        off = cidx * CH
        ca = pltpu.async_copy(tbl.at[idx1_v.at[pl.ds(off, CH)]], rows_a, sem_a)
        cb = pltpu.async_copy(tbl.at[idx2_v.at[pl.ds(off, CH)]], rows_b, sem_b)
        cw = pltpu.async_copy(w.at[pl.ds((base + off) * C, CH * C)], w_v, sem_w)
        ca.wait()
        cb.wait()
        cw.wait()

        skew = lane * 17

        def group_body(g, gcarry):
            tok = lane + g * LANES

            def j_body(s, jc):
                mx, mi = jc
                jv = (skew + s) & (C - 1)
                a = plsc.load_gather(rows_a, [tok, jv])
                b = plsc.load_gather(rows_b, [tok, jv])
                wv = plsc.load_gather(w_v, [tok * C + jv])
                v = (a + EPS) * (b + EPS) * wv
                upd = v > mx
                mx = jnp.where(upd, v, mx)
                mi = jnp.where(upd, jv, mi)
                return (mx, mi)

            _, mi = lax.fori_loop(
                0, C, j_body,
                (jnp.full((LANES,), -1.0, jnp.float32),
                 jnp.zeros((LANES,), jnp.int32)),
                unroll=8)
            sl = pl.ds(off + g * LANES, LANES)
            out_v[sl] = jnp.where(tt_v[sl] == T + 1, xe_v[sl], mi)
            return gcarry

        lax.fori_loop(0, CH // LANES, group_body, 0)
        return carry

    lax.fori_loop(0, NCHUNK, chunk_body, 0)
    pltpu.sync_copy(out_v, out.at[pl.ds(base, NPW)])


_sc_call = pl.kernel(
    _sc_body,
    out_type=jax.ShapeDtypeStruct((NTOK,), jnp.int32),
    mesh=plsc.VectorSubcoreMesh(core_axis_name="c", subcore_axis_name="s"),
    compiler_params=pltpu.CompilerParams(use_tc_tiling_on_sc=False,
                                         needs_layout_passes=False),
    scratch_types=[
        pltpu.VMEM((NPW,), jnp.int32),
        pltpu.VMEM((NPW,), jnp.int32),
        pltpu.VMEM((NPW,), jnp.int32),
        pltpu.VMEM((NPW,), jnp.int32),
        pltpu.VMEM((NPW,), jnp.int32),
        pltpu.VMEM((CH, C), jnp.float32),
        pltpu.VMEM((CH, C), jnp.float32),
        pltpu.VMEM((CH * C,), jnp.float32),
        pltpu.VMEM((NPW,), jnp.int32),
        pltpu.SemaphoreType.DMA,
        pltpu.SemaphoreType.DMA,
        pltpu.SemaphoreType.DMA,
    ],
)


_K1, _K2 = (int(x) for x in np.asarray(
    jax.random.key_data(jax.random.fold_in(jax.random.key(0), 1))))


def _gumbel_weights(lo, hi):
    c2 = lax.iota(jnp.uint32, hi - lo) + np.uint32(lo)
    b1, b2 = threefry2x32_p.bind(
        jnp.asarray(_K1, jnp.uint32), jnp.asarray(_K2, jnp.uint32),
        jnp.zeros((hi - lo,), jnp.uint32), c2)
    fb = ((b1 ^ b2) >> np.uint32(9)) | np.uint32(0x3F800000)
    u = lax.bitcast_convert_type(fb, jnp.float32) - 1.0
    u = jnp.clip(u, jnp.finfo(jnp.float32).tiny, 1.0)
    return 1.0 / (-jnp.log(u))


def kernel(x_start, x_end, t, p_onestep, p_cum):
    tbl = p_cum.reshape(NT * C, C)
    xs = x_start.reshape(N)
    xe = x_end.reshape(N)
    tt = jnp.repeat(t, L)
    outs = []
    for s in range(STAGES):
        a = s * NTOK
        w_s = _gumbel_weights(a * C, (a + NTOK) * C)
        outs.append(_sc_call(tbl, xs[a:a + NTOK], xe[a:a + NTOK],
                             tt[a:a + NTOK], w_s))
    return jnp.concatenate(outs).reshape(B, L)


# speedup vs baseline: 1.0198x; 1.0018x over previous
import jax
import jax.numpy as jnp
import numpy as np
from jax import lax
from jax._src.random.threefry2x32 import threefry2x32_p
from jax.experimental import pallas as pl
from jax.experimental.pallas import tpu as pltpu
from jax.experimental.pallas import tpu_sc as plsc

C = 256
T = 500
NT = T + 2
EPS = 1e-6
B, L = 4096, 50
N = B * L
NC, NS = 2, 16
NW = NC * NS
STAGES = 8
NTOK = N // STAGES
NPW = NTOK // NW
CH = 80
NCHUNK = NPW // CH
assert NTOK * STAGES == N and NPW * NW == NTOK and NCHUNK * CH == NPW
assert CH % 8 == 0 and CH <= 128
LANES = 16
JC = C // LANES


def _sc_body(tbl, xs, xe, tt, w, out,
             xs_v, xe_v, tt_v, idx1_v, idx2_v,
             rows_a, rows_b, w_v, out_v,
             sem_a, sem_b, sem_w):
    wid = lax.axis_index("s") * NC + lax.axis_index("c")
    base = wid * NPW

    pltpu.sync_copy(xs.at[pl.ds(base, NPW)], xs_v)
    pltpu.sync_copy(xe.at[pl.ds(base, NPW)], xe_v)
    pltpu.sync_copy(tt.at[pl.ds(base, NPW)], tt_v)

    def idx_body(k, carry):
        sl = pl.ds(k * LANES, LANES)
        tv = tt_v[sl]
        idx1_v[sl] = tv * C + xs_v[sl]
        idx2_v[sl] = (T + 1 - tv) * C + xe_v[sl]
        return carry

    lax.fori_loop(0, NPW // LANES, idx_body, 0)

    lane = lax.iota(jnp.int32, LANES)

    def chunk_body(cidx, carry):
        off = cidx * CH
        ca = pltpu.async_copy(tbl.at[idx1_v.at[pl.ds(off, CH)]], rows_a, sem_a)
        cb = pltpu.async_copy(tbl.at[idx2_v.at[pl.ds(off, CH)]], rows_b, sem_b)
        cw = pltpu.async_copy(w.at[pl.ds((base + off) * C, CH * C)], w_v, sem_w)
        ca.wait()
        cb.wait()
        cw.wait()

        skew = lane * 17

        def group_body(g, gcarry):
            tok = lane + g * LANES

            def j_body(s, jc):
                bp, bnl, mi = jc
                jv = (skew + s) & (C - 1)
                a = plsc.load_gather(rows_a, [tok, jv])
                b = plsc.load_gather(rows_b, [tok, jv])
                nl = plsc.load_gather(w_v, [tok * C + jv])
                p = (a + EPS) * (b + EPS)
                upd = p * bnl > bp * nl
                bp = jnp.where(upd, p, bp)
                bnl = jnp.where(upd, nl, bnl)
                mi = jnp.where(upd, jv, mi)
                return (bp, bnl, mi)

            _, _, mi = lax.fori_loop(
                0, C, j_body,
                (jnp.full((LANES,), -1.0, jnp.float32),
                 jnp.full((LANES,), 1.0, jnp.float32),
                 jnp.zeros((LANES,), jnp.int32)),
                unroll=8)
            sl = pl.ds(off + g * LANES, LANES)
            out_v[sl] = jnp.where(tt_v[sl] == T + 1, xe_v[sl], mi)
            return gcarry

        lax.fori_loop(0, CH // LANES, group_body, 0)
        return carry

    lax.fori_loop(0, NCHUNK, chunk_body, 0)
    pltpu.sync_copy(out_v, out.at[pl.ds(base, NPW)])


_sc_call = pl.kernel(
    _sc_body,
    out_type=jax.ShapeDtypeStruct((NTOK,), jnp.int32),
    mesh=plsc.VectorSubcoreMesh(core_axis_name="c", subcore_axis_name="s"),
    compiler_params=pltpu.CompilerParams(use_tc_tiling_on_sc=False,
                                         needs_layout_passes=False),
    scratch_types=[
        pltpu.VMEM((NPW,), jnp.int32),
        pltpu.VMEM((NPW,), jnp.int32),
        pltpu.VMEM((NPW,), jnp.int32),
        pltpu.VMEM((NPW,), jnp.int32),
        pltpu.VMEM((NPW,), jnp.int32),
        pltpu.VMEM((CH, C), jnp.float32),
        pltpu.VMEM((CH, C), jnp.float32),
        pltpu.VMEM((CH * C,), jnp.float32),
        pltpu.VMEM((NPW,), jnp.int32),
        pltpu.SemaphoreType.DMA,
        pltpu.SemaphoreType.DMA,
        pltpu.SemaphoreType.DMA,
    ],
)


_K1, _K2 = (int(x) for x in np.asarray(
    jax.random.key_data(jax.random.fold_in(jax.random.key(0), 1))))


def _neglog_noise(lo, hi):
    c2 = lax.iota(jnp.uint32, hi - lo) + np.uint32(lo)
    b1, b2 = threefry2x32_p.bind(
        jnp.asarray(_K1, jnp.uint32), jnp.asarray(_K2, jnp.uint32),
        jnp.zeros((hi - lo,), jnp.uint32), c2)
    fb = ((b1 ^ b2) >> np.uint32(9)) | np.uint32(0x3F800000)
    u = lax.bitcast_convert_type(fb, jnp.float32) - 1.0
    u = jnp.clip(u, jnp.finfo(jnp.float32).tiny, 1.0)
    return -jnp.log(u)


def kernel(x_start, x_end, t, p_onestep, p_cum):
    tbl = p_cum.reshape(NT * C, C)
    xs = x_start.reshape(N)
    xe = x_end.reshape(N)
    tt = jnp.repeat(t, L)
    outs = []
    for s in range(STAGES):
        a = s * NTOK
        w_s = _neglog_noise(a * C, (a + NTOK) * C)
        outs.append(_sc_call(tbl, xs[a:a + NTOK], xe[a:a + NTOK],
                             tt[a:a + NTOK], w_s))
    return jnp.concatenate(outs).reshape(B, L)
